# baseline (speedup 1.0000x reference)
import math

import jax
import jax.numpy as jnp
from jax import lax
from jax.experimental import pallas as pl
from jax.experimental.pallas import tpu as pltpu
from jax.experimental.pallas import tpu_sc as plsc

D_MODEL = 64
SCALE = math.sqrt(D_MODEL)
NC, NS = 2, 16
NW = NC * NS
CHUNK = 128
NBUF = 4
LOOKAHEAD = 2


def _scale_chunk(buf):
    @plsc.parallel_loop(0, CHUNK, step=1, unroll=8)
    def _(i):
        for j in range(D_MODEL // 16):
            buf[i, pl.ds(j * 16, 16)] = buf[i, pl.ds(j * 16, 16)] * SCALE


def _embed_body(x_hbm, table_hbm, out_hbm, idx_v, bufs, gsems, ssems):
    wid = lax.axis_index("s") * NC + lax.axis_index("c")
    per_w = x_hbm.shape[0] // NW
    nchunk = per_w // CHUNK
    base = wid * per_w

    pltpu.sync_copy(x_hbm.at[pl.ds(base, per_w)], idx_v)

    def start_gather(g, b):
        pltpu.async_copy(
            table_hbm.at[idx_v.at[pl.ds(g * CHUNK, CHUNK)]], bufs[b], gsems[b])

    def wait_gather(g, b):
        pltpu.make_async_copy(
            table_hbm.at[idx_v.at[pl.ds(g * CHUNK, CHUNK)]], bufs[b], gsems[b]).wait()

    def start_scatter(g, b):
        pltpu.async_copy(
            bufs[b], out_hbm.at[pl.ds(base + g * CHUNK, CHUNK)], ssems[b])

    def wait_scatter(g, b):
        pltpu.make_async_copy(
            bufs[b], out_hbm.at[pl.ds(base + g * CHUNK, CHUNK)], ssems[b]).wait()

    for k in range(LOOKAHEAD):
        start_gather(k, k)

    def outer(o, cur):
        for b in range(NBUF):
            g = o * NBUF + b
            pb = (b + LOOKAHEAD) % NBUF
            gp = g + LOOKAHEAD

            @pl.when(jnp.logical_and(gp < nchunk, gp >= NBUF))
            def _():
                wait_scatter(gp - NBUF, pb)

            @pl.when(gp < nchunk)
            def _():
                start_gather(gp, pb)

            wait_gather(g, b)
            _scale_chunk(bufs[b])
            start_scatter(g, b)
        return cur

    lax.fori_loop(0, nchunk // NBUF, outer, jnp.int32(0))

    for b in range(NBUF):
        wait_scatter(nchunk - NBUF + b, b)


def kernel(x, table):
    x_flat = x.reshape(-1).astype(jnp.int32)
    b_total = x_flat.shape[0]
    mesh = plsc.VectorSubcoreMesh(core_axis_name="c", subcore_axis_name="s")
    per_w = b_total // NW

    def body(x_hbm, table_hbm, out_hbm,
             idx_v, b0, b1, b2, b3, g0, g1, g2, g3, s0, s1, s2, s3):
        _embed_body(x_hbm, table_hbm, out_hbm, idx_v,
                    (b0, b1, b2, b3), (g0, g1, g2, g3), (s0, s1, s2, s3))

    out = pl.kernel(
        body,
        out_type=jax.ShapeDtypeStruct((b_total, D_MODEL), jnp.float32),
        mesh=mesh,
        compiler_params=pltpu.CompilerParams(use_tc_tiling_on_sc=False),
        scratch_types=[
            pltpu.VMEM((per_w,), jnp.int32),
        ] + [pltpu.VMEM((CHUNK, D_MODEL), jnp.float32)] * NBUF
          + [pltpu.SemaphoreType.DMA] * (2 * NBUF),
    )(x_flat, table)
    return out.reshape(x.shape + (D_MODEL,))
